# baseline (speedup 1.0000x reference)
import functools

import jax
import jax.numpy as jnp
from jax import lax
from jax.experimental import pallas as pl
from jax.experimental.pallas import tpu as pltpu
from jax.experimental.pallas import tpu_sc as plsc

_NBUF = 7
_CHUNK_ROWS = 16


def _make_copy_kernel(seq_len: int, n_embd: int):
    info = plsc.get_sparse_core_info()
    nc, ns = info.num_cores, info.num_subcores
    nw = nc * ns
    assert seq_len % nw == 0
    rows_per_w = seq_len // nw
    assert rows_per_w % _CHUNK_ROWS == 0
    n_chunks = rows_per_w // _CHUNK_ROWS
    mesh = plsc.VectorSubcoreMesh(core_axis_name="c", subcore_axis_name="s")

    @functools.partial(
        pl.kernel,
        mesh=mesh,
        out_type=jax.ShapeDtypeStruct((seq_len, n_embd), jnp.float32),
        scratch_types=[
            pltpu.VMEM((_NBUF, _CHUNK_ROWS, n_embd), jnp.float32),
            pltpu.SemaphoreType.DMA((_NBUF,)),
            pltpu.SemaphoreType.DMA((_NBUF,)),
        ],
    )
    def copy_kernel(emb_hbm, out_hbm, buf, in_sems, out_sems):
        wid = lax.axis_index("s") * nc + lax.axis_index("c")
        base = wid * rows_per_w

        def in_copy(i, b):
            return pltpu.make_async_copy(
                emb_hbm.at[pl.ds(base + i * _CHUNK_ROWS, _CHUNK_ROWS)],
                buf.at[b],
                in_sems.at[b],
            )

        def out_copy(i, b):
            return pltpu.make_async_copy(
                buf.at[b],
                out_hbm.at[pl.ds(base + i * _CHUNK_ROWS, _CHUNK_ROWS)],
                out_sems.at[b],
            )

        for i in range(min(_NBUF, n_chunks)):
            in_copy(i, i).start()
        for i in range(n_chunks):
            b = i % _NBUF
            in_copy(i, b).wait()
            out_copy(i, b).start()
            nxt = i + _NBUF
            if nxt < n_chunks:
                out_copy(i, b).wait()
                in_copy(nxt, b).start()
        for i in range(max(n_chunks - _NBUF, 0), n_chunks):
            out_copy(i, i % _NBUF).wait()

    return copy_kernel


def kernel(x, emb):
    seq_len = x.shape[1]
    return _make_copy_kernel(seq_len, emb.shape[1])(emb)
